# baseline (speedup 1.0000x reference)
import functools

import jax
import jax.numpy as jnp
from jax import lax
from jax.experimental import pallas as pl
from jax.experimental.pallas import tpu as pltpu
from jax.experimental.pallas import tpu_sc as plsc

_CHUNK = 128


@functools.lru_cache(maxsize=None)
def _build_gather(num_rows: int, dim: int, batch: int):
    info = plsc.get_sparse_core_info()
    nc, ns = info.num_cores, info.num_subcores
    nw = nc * ns
    assert batch % (nw * _CHUNK) == 0
    chunks_per_w = batch // (nw * _CHUNK)

    mesh = plsc.VectorSubcoreMesh(core_axis_name="c", subcore_axis_name="s")

    @functools.partial(
        pl.kernel,
        mesh=mesh,
        compiler_params=pltpu.CompilerParams(use_tc_tiling_on_sc=False),
        out_type=jax.ShapeDtypeStruct((batch, dim), jnp.float32),
        scratch_types=[
            pltpu.VMEM((chunks_per_w, _CHUNK), jnp.int32),
            pltpu.VMEM((_CHUNK, dim), jnp.float32),
            pltpu.SemaphoreType.DMA,
        ],
    )
    def gather_kernel(table_hbm, idx_hbm, out_hbm, idx_v, rows_v, sem):
        wid = lax.axis_index("s") * nc + lax.axis_index("c")
        pltpu.sync_copy(idx_hbm.at[pl.ds(wid * chunks_per_w, chunks_per_w)], idx_v)

        def chunk(j, carry):
            pltpu.async_copy(table_hbm.at[idx_v.at[j]], rows_v, sem).wait()
            base = (wid * chunks_per_w + j) * _CHUNK
            pltpu.sync_copy(rows_v, out_hbm.at[pl.ds(base, _CHUNK)])
            return carry

        lax.fori_loop(0, chunks_per_w, chunk, 0)

    return gather_kernel


def kernel(token_ids, weight):
    b, s = token_ids.shape
    num_rows, dim = weight.shape
    batch = b * s
    idx2d = token_ids.reshape(batch // _CHUNK, _CHUNK).astype(jnp.int32)
    out = _build_gather(num_rows, dim, batch)(weight, idx2d)
    return out.reshape(b, s, dim)


# speedup vs baseline: 1.1123x; 1.1123x over previous
import functools

import jax
import jax.numpy as jnp
from jax import lax
from jax.experimental import pallas as pl
from jax.experimental.pallas import tpu as pltpu
from jax.experimental.pallas import tpu_sc as plsc

_CHUNK = 128
_CPG = 4


@functools.lru_cache(maxsize=None)
def _build_gather(num_rows: int, dim: int, batch: int):
    info = plsc.get_sparse_core_info()
    nc, ns = info.num_cores, info.num_subcores
    nw = nc * ns
    rows_g = _CPG * _CHUNK
    assert batch % (nw * rows_g) == 0
    chunks_per_w = batch // (nw * _CHUNK)
    groups = chunks_per_w // _CPG
    assert groups % 2 == 0

    mesh = plsc.VectorSubcoreMesh(core_axis_name="c", subcore_axis_name="s")

    @functools.partial(
        pl.kernel,
        mesh=mesh,
        compiler_params=pltpu.CompilerParams(use_tc_tiling_on_sc=False),
        out_type=jax.ShapeDtypeStruct((batch, dim), jnp.float32),
        scratch_types=[
            pltpu.VMEM((chunks_per_w, _CHUNK), jnp.int32),
            pltpu.VMEM((rows_g, dim), jnp.float32),
            pltpu.VMEM((rows_g, dim), jnp.float32),
            pltpu.SemaphoreType.DMA,
            pltpu.SemaphoreType.DMA,
            pltpu.SemaphoreType.DMA,
            pltpu.SemaphoreType.DMA,
        ],
    )
    def gather_kernel(table_hbm, idx_hbm, out_hbm, idx_v, buf0, buf1,
                      gsem0, gsem1, ssem0, ssem1):
        bufs = (buf0, buf1)
        gsems = (gsem0, gsem1)
        ssems = (ssem0, ssem1)
        wid = lax.axis_index("s") * nc + lax.axis_index("c")
        row0 = wid * chunks_per_w * _CHUNK
        pltpu.sync_copy(idx_hbm.at[pl.ds(wid * chunks_per_w, chunks_per_w)], idx_v)

        def fire(g, slot):
            for k in range(_CPG):
                pltpu.async_copy(
                    table_hbm.at[idx_v.at[g * _CPG + k]],
                    bufs[slot].at[pl.ds(k * _CHUNK, _CHUNK)],
                    gsems[slot])

        def wait_gathers(slot):
            pltpu.make_async_copy(
                out_hbm.at[pl.ds(0, rows_g)], bufs[slot], gsems[slot]).wait()

        def start_store(g, slot):
            pltpu.async_copy(
                bufs[slot], out_hbm.at[pl.ds(row0 + g * rows_g, rows_g)],
                ssems[slot])

        def wait_store(slot):
            pltpu.make_async_copy(
                bufs[slot], out_hbm.at[pl.ds(0, rows_g)], ssems[slot]).wait()

        fire(0, 0)

        def body(t, carry):
            g0 = 2 * t
            wait_gathers(0)
            start_store(g0, 0)

            @pl.when(t > 0)
            def _():
                wait_store(1)
            fire(g0 + 1, 1)
            wait_gathers(1)
            start_store(g0 + 1, 1)
            wait_store(0)

            @pl.when(t < groups // 2 - 1)
            def _():
                fire(g0 + 2, 0)
            return carry

        lax.fori_loop(0, groups // 2, body, 0)
        wait_store(1)

    return gather_kernel


def kernel(token_ids, weight):
    b, s = token_ids.shape
    num_rows, dim = weight.shape
    batch = b * s
    idx2d = token_ids.reshape(batch // _CHUNK, _CHUNK).astype(jnp.int32)
    out = _build_gather(num_rows, dim, batch)(weight, idx2d)
    return out.reshape(b, s, dim)


# speedup vs baseline: 1.3581x; 1.2210x over previous
import functools

import jax
import jax.numpy as jnp
from jax import lax
from jax.experimental import pallas as pl
from jax.experimental.pallas import tpu as pltpu
from jax.experimental.pallas import tpu_sc as plsc

_CHUNK = 128
_CPG = 2
_NSLOT = 3
_PAD = 128


@functools.lru_cache(maxsize=None)
def _build_gather(num_rows: int, dim: int, batch: int):
    info = plsc.get_sparse_core_info()
    nc, ns = info.num_cores, info.num_subcores
    nw = nc * ns
    rows_g = _CPG * _CHUNK
    assert batch % (nw * rows_g) == 0
    chunks_per_w = batch // (nw * _CHUNK)
    groups = chunks_per_w // _CPG
    assert groups % _NSLOT == 1 and groups > 2 * _NSLOT

    mesh = plsc.VectorSubcoreMesh(core_axis_name="c", subcore_axis_name="s")

    @functools.partial(
        pl.kernel,
        mesh=mesh,
        out_type=jax.ShapeDtypeStruct((batch, _PAD), jnp.float32),
        scratch_types=[
            pltpu.VMEM((chunks_per_w, _CHUNK), jnp.int32),
            pltpu.VMEM((rows_g, _PAD), jnp.float32),
            pltpu.VMEM((rows_g, _PAD), jnp.float32),
            pltpu.VMEM((rows_g, _PAD), jnp.float32),
            pltpu.SemaphoreType.DMA,
            pltpu.SemaphoreType.DMA,
            pltpu.SemaphoreType.DMA,
            pltpu.SemaphoreType.DMA,
            pltpu.SemaphoreType.DMA,
            pltpu.SemaphoreType.DMA,
        ],
    )
    def gather_kernel(table_hbm, idx_hbm, out_hbm, idx_v, buf0, buf1, buf2,
                      gsem0, gsem1, gsem2, ssem0, ssem1, ssem2):
        bufs = (buf0, buf1, buf2)
        gsems = (gsem0, gsem1, gsem2)
        ssems = (ssem0, ssem1, ssem2)
        wid = lax.axis_index("s") * nc + lax.axis_index("c")
        row0 = wid * chunks_per_w * _CHUNK
        pltpu.sync_copy(idx_hbm.at[pl.ds(wid * chunks_per_w, chunks_per_w)], idx_v)

        def fire(g, slot):
            for k in range(_CPG):
                pltpu.async_copy(
                    table_hbm.at[idx_v.at[g * _CPG + k]],
                    bufs[slot].at[pl.ds(k * _CHUNK, _CHUNK)],
                    gsems[slot])

        def wait_gathers(slot):
            pltpu.make_async_copy(
                table_hbm.at[pl.ds(0, rows_g)], bufs[slot], gsems[slot]).wait()

        def start_store(g, slot):
            pltpu.async_copy(
                bufs[slot], out_hbm.at[pl.ds(row0 + g * rows_g, rows_g)],
                ssems[slot])

        def wait_store(slot):
            pltpu.make_async_copy(
                bufs[slot], out_hbm.at[pl.ds(0, rows_g)], ssems[slot]).wait()

        def step(g, slot, first, fire_next):
            wait_gathers(slot)
            start_store(g, slot)
            if not first:
                wait_store((slot + 2) % _NSLOT)
            if fire_next is not None:
                fire_next()

        fire(0, 0)
        fire(1, 1)
        step(0, 0, True, lambda: fire(2, 2))
        step(1, 1, False, lambda: fire(3, 0))
        step(2, 2, False, lambda: fire(4, 1))

        def body(t, carry):
            for u in range(_NSLOT):
                g = _NSLOT * t + u

                def fire_next(g=g, u=u):
                    @pl.when(g + 2 < groups)
                    def _():
                        fire(g + 2, (u + 2) % _NSLOT)

                step(g, u, False, fire_next)
            return carry

        lax.fori_loop(1, groups // _NSLOT, body, 0)
        g_last = groups - 1
        step(g_last, g_last % _NSLOT, False, None)
        wait_store(g_last % _NSLOT)

    return gather_kernel


def kernel(token_ids, weight):
    b, s = token_ids.shape
    num_rows, dim = weight.shape
    batch = b * s
    idx2d = token_ids.reshape(batch // _CHUNK, _CHUNK).astype(jnp.int32)
    wpad = jnp.pad(weight, ((0, 0), (0, _PAD - dim)))
    out = _build_gather(num_rows, dim, batch)(wpad, idx2d)
    return out.reshape(b, s, _PAD)[:, :, :dim]
